# baseline (speedup 1.0000x reference)
import functools
import math

import jax
import jax.numpy as jnp
from jax import lax
from jax.experimental import pallas as pl
from jax.experimental.pallas import tpu as pltpu
from jax.experimental.pallas import tpu_sc as plsc

_VOCAB = 1000000
_HIDDEN = 64
_B = 1024
_L = 200
_SCALE = math.sqrt(_HIDDEN)

_NC = 2
_NS = 16
_NW = _NC * _NS
_SEQ_PER_W = _B // _NW

_IA = 104
_IB = 96


def _body(source_hbm, token_hbm, pos_hbm, out_hbm,
          pos_v, idx_a, idx_b, tok_v, sem):
  wid = lax.axis_index("s") * _NC + lax.axis_index("c")

  pltpu.sync_copy(pos_hbm.at[pl.ds(0, _L)], pos_v)

  def seq_body(s, carry):
    seq = wid * _SEQ_PER_W + s
    base = pl.multiple_of(seq * _L, 8)
    pltpu.sync_copy(source_hbm.at[pl.ds(base, _IA)], idx_a)
    pltpu.sync_copy(source_hbm.at[pl.ds(base + _IA, _IB)], idx_b)
    cp1 = pltpu.async_copy(token_hbm.at[idx_a], tok_v.at[pl.ds(0, _IA)], sem)
    cp2 = pltpu.async_copy(token_hbm.at[idx_b], tok_v.at[pl.ds(_IA, _IB)], sem)
    cp1.wait()
    cp2.wait()

    def row_body(r, c2):
      for c in range(_HIDDEN // 16):
        sl = pl.ds(c * 16, 16)
        tok_v[r, sl] = tok_v[r, sl] * _SCALE + pos_v[r, sl]
      return c2

    lax.fori_loop(0, _L, row_body, 0)
    pltpu.sync_copy(tok_v, out_hbm.at[seq])
    return carry

  lax.fori_loop(0, _SEQ_PER_W, seq_body, 0)


@functools.partial(jax.jit, static_argnames=())
def kernel(source, token_table, pos_table):
  mesh = plsc.VectorSubcoreMesh(core_axis_name="c", subcore_axis_name="s",
                                num_cores=_NC, num_subcores=_NS)
  run = pl.kernel(
      _body,
      out_type=jax.ShapeDtypeStruct((_B, _L, _HIDDEN), jnp.float32),
      mesh=mesh,
      scratch_types=[
          pltpu.VMEM((_L, _HIDDEN), jnp.float32),
          pltpu.VMEM((_IA,), jnp.int32),
          pltpu.VMEM((_IB,), jnp.int32),
          pltpu.VMEM((_L, _HIDDEN), jnp.float32),
          pltpu.SemaphoreType.DMA,
      ],
      compiler_params=pltpu.CompilerParams(use_tc_tiling_on_sc=False),
  )
  return run(source.reshape(-1), token_table, pos_table)
